# baseline (device time: 184122 ns/iter reference)
import functools

import jax
import jax.numpy as jnp
from jax import lax
from jax.experimental import pallas as pl
from jax.experimental.pallas import tpu as pltpu

B, S, H, Dh, Dr = 4, 256, 32, 128, 64
D = 4096
DC = 256
DC_SH = 128
M = B * S

_CompilerParams = getattr(pltpu, "CompilerParams", None) or getattr(
    pltpu, "TPUCompilerParams"
)
_VMEM_LIMIT = 100 * 1024 * 1024


_QBLK = 512


def _exq_body(
    x_ref, wdkv_ref, wuk_ref, wuv_ref, wq_ref,
    xbf_ref, c_ref, wuk2_ref, wuv2_ref, q_ref,
    c_loc, wukbf, wuvbf,
    c_rem, wuk_rem, wuv_rem,
    send_sems, recv_sems,
):
    j = pl.program_id(0)
    my_x = lax.axis_index("x")
    my_y = lax.axis_index("y")
    my_z = lax.axis_index("z")
    partner = (1 - my_x, my_y, my_z)

    rdmas = [
        pltpu.make_async_remote_copy(
            src_ref=src,
            dst_ref=dst,
            send_sem=send_sems.at[i],
            recv_sem=recv_sems.at[i],
            device_id=partner,
            device_id_type=pl.DeviceIdType.MESH,
        )
        for i, (src, dst) in enumerate(
            [(wukbf, wuk_rem), (wuvbf, wuv_rem), (c_loc, c_rem)]
        )
    ]

    @pl.when(j == 0)
    def _():
        barrier = pltpu.get_barrier_semaphore()
        pl.semaphore_signal(
            barrier, inc=1, device_id=partner,
            device_id_type=pl.DeviceIdType.MESH,
        )
        pl.semaphore_wait(barrier, 1)

        wukbf[...] = wuk_ref[...].astype(jnp.bfloat16)
        wuvbf[...] = wuv_ref[...].astype(jnp.bfloat16)
        rdmas[0].start()
        rdmas[1].start()
        xbf_ref[...] = x_ref[...].astype(jnp.bfloat16)
        c_loc[...] = jnp.dot(
            xbf_ref[...], wdkv_ref[...].astype(jnp.bfloat16),
            preferred_element_type=jnp.float32,
        ).astype(jnp.bfloat16)
        rdmas[2].start()
        c_ref[:, 0:DC_SH] = c_loc[...]
        wuk2_ref[0:DC_SH, :] = wukbf[...]
        wuv2_ref[0:DC_SH, :] = wuvbf[...]

    q_ref[...] = jnp.dot(
        xbf_ref[...], wq_ref[...].astype(jnp.bfloat16),
        preferred_element_type=jnp.float32,
    ).astype(jnp.bfloat16)

    @pl.when(j == pl.num_programs(0) - 1)
    def _():
        for rdma in rdmas:
            rdma.wait()
        c_ref[:, DC_SH:DC] = c_rem[...]
        wuk2_ref[DC_SH:DC, :] = wuk_rem[...]
        wuv2_ref[DC_SH:DC, :] = wuv_rem[...]


def _exchange_q(x2, wdkv, wuk, wuv, wq):
    return pl.pallas_call(
        _exq_body,
        grid=(D // _QBLK,),
        out_shape=[
            jax.ShapeDtypeStruct((M, D), jnp.bfloat16),
            jax.ShapeDtypeStruct((M, DC), jnp.bfloat16),
            jax.ShapeDtypeStruct((DC, D), jnp.bfloat16),
            jax.ShapeDtypeStruct((DC, D), jnp.bfloat16),
            jax.ShapeDtypeStruct((M, D), jnp.bfloat16),
        ],
        in_specs=[
            pl.BlockSpec((M, D), lambda j: (0, 0)),
            pl.BlockSpec((D, DC_SH), lambda j: (0, 0)),
            pl.BlockSpec((DC_SH, D), lambda j: (0, 0)),
            pl.BlockSpec((DC_SH, D), lambda j: (0, 0)),
            pl.BlockSpec((D, _QBLK), lambda j: (0, j)),
        ],
        out_specs=[
            pl.BlockSpec((M, D), lambda j: (0, 0)),
            pl.BlockSpec((M, DC), lambda j: (0, 0)),
            pl.BlockSpec((DC, D), lambda j: (0, 0)),
            pl.BlockSpec((DC, D), lambda j: (0, 0)),
            pl.BlockSpec((M, _QBLK), lambda j: (0, j)),
        ],
        scratch_shapes=[
            pltpu.VMEM((M, DC_SH), jnp.bfloat16),
            pltpu.VMEM((DC_SH, D), jnp.bfloat16),
            pltpu.VMEM((DC_SH, D), jnp.bfloat16),
            pltpu.VMEM((M, DC_SH), jnp.bfloat16),
            pltpu.VMEM((DC_SH, D), jnp.bfloat16),
            pltpu.VMEM((DC_SH, D), jnp.bfloat16),
            pltpu.SemaphoreType.DMA((3,)),
            pltpu.SemaphoreType.DMA((3,)),
        ],
        compiler_params=_CompilerParams(
            collective_id=0, vmem_limit_bytes=_VMEM_LIMIT
        ),
    )(x2, wdkv, wuk, wuv, wq)


def _matmul_body(a_ref, w_ref, o_ref, *, out_dtype):
    a = a_ref[...]
    w = w_ref[...]
    if a.dtype != jnp.bfloat16:
        a = a.astype(jnp.bfloat16)
    if w.dtype != jnp.bfloat16:
        w = w.astype(jnp.bfloat16)
    o_ref[...] = jnp.dot(a, w, preferred_element_type=jnp.float32).astype(
        out_dtype
    )


def _matmul(a, w, bn, out_dtype):
    m, k = a.shape
    _, n = w.shape
    assert n % bn == 0
    return pl.pallas_call(
        functools.partial(_matmul_body, out_dtype=out_dtype),
        grid=(n // bn,),
        in_specs=[
            pl.BlockSpec((m, k), lambda j: (0, 0)),
            pl.BlockSpec((k, bn), lambda j: (0, j)),
        ],
        out_specs=pl.BlockSpec((m, bn), lambda j: (0, j)),
        out_shape=jax.ShapeDtypeStruct((m, n), out_dtype),
        compiler_params=_CompilerParams(vmem_limit_bytes=_VMEM_LIMIT),
    )(a, w)


def _qr_kr_body(a_ref, wqr_ref, wkr_ref, qr_ref, kr_ref):
    @pl.when(pl.program_id(0) == 0)
    def _():
        kr_ref[...] = jnp.dot(
            a_ref[...], wkr_ref[...].astype(jnp.bfloat16),
            preferred_element_type=jnp.float32,
        ).astype(jnp.bfloat16)

    qr_ref[...] = jnp.dot(
        a_ref[...], wqr_ref[...].astype(jnp.bfloat16),
        preferred_element_type=jnp.float32,
    ).astype(jnp.bfloat16)


def _qr_kr(a, wqr, wkr):
    bn = 512
    n = H * Dr
    return pl.pallas_call(
        _qr_kr_body,
        grid=(n // bn,),
        in_specs=[
            pl.BlockSpec((M, D), lambda j: (0, 0)),
            pl.BlockSpec((D, bn), lambda j: (0, j)),
            pl.BlockSpec((D, Dr), lambda j: (0, 0)),
        ],
        out_specs=[
            pl.BlockSpec((M, bn), lambda j: (0, j)),
            pl.BlockSpec((M, Dr), lambda j: (0, 0)),
        ],
        out_shape=[
            jax.ShapeDtypeStruct((M, n), jnp.bfloat16),
            jax.ShapeDtypeStruct((M, Dr), jnp.bfloat16),
        ],
        compiler_params=_CompilerParams(vmem_limit_bytes=_VMEM_LIMIT),
    )(a, wqr, wkr)


def _attn_body(c_ref, wuk_ref, wuv_ref, q_ref, qr_ref, kr_ref, o_ref,
               k_s, v_s):
    k_s[...] = jnp.dot(
        c_ref[...], wuk_ref[...], preferred_element_type=jnp.float32
    ).astype(jnp.bfloat16)
    v_s[...] = jnp.dot(
        c_ref[...], wuv_ref[...], preferred_element_type=jnp.float32
    ).astype(jnp.bfloat16)

    kr = kr_ref[...]
    kr_cat = jnp.concatenate([kr, kr], axis=1)
    lane = lax.broadcasted_iota(jnp.int32, (S, 2 * Dr), 1)
    scale = (Dh + Dr) ** -0.5
    masks = [(lane // Dr) == 0, (lane // Dr) == 1]
    dims = (((1,), (1,)), ((), ()))
    for h in range(H):
        q = (q_ref[:, h * Dh:(h + 1) * Dh] * scale).astype(jnp.bfloat16)
        k = k_s[:, h * Dh:(h + 1) * Dh]
        v = v_s[:, h * Dh:(h + 1) * Dh]
        qrp = qr_ref[:, (h // 2) * 2 * Dr:(h // 2 + 1) * 2 * Dr]
        qr_m = jnp.where(masks[h % 2], qrp * scale, 0).astype(jnp.bfloat16)
        s = lax.dot_general(q, k, dims, preferred_element_type=jnp.float32)
        s = s + lax.dot_general(
            qr_m, kr_cat, dims, preferred_element_type=jnp.float32
        )
        p = jnp.exp(s)
        denom = jnp.sum(p, axis=1, keepdims=True)
        o = jnp.dot(
            p.astype(jnp.bfloat16), v, preferred_element_type=jnp.float32
        )
        o_ref[:, h * Dh:(h + 1) * Dh] = (o * (1.0 / denom)).astype(
            jnp.bfloat16
        )


def _attention(c2, wuk2, wuv2, q2, qr2, kr2):
    return pl.pallas_call(
        _attn_body,
        grid=(B,),
        in_specs=[
            pl.BlockSpec((S, DC), lambda b: (b, 0)),
            pl.BlockSpec((DC, D), lambda b: (0, 0)),
            pl.BlockSpec((DC, D), lambda b: (0, 0)),
            pl.BlockSpec((S, D), lambda b: (b, 0)),
            pl.BlockSpec((S, H * Dr), lambda b: (b, 0)),
            pl.BlockSpec((S, Dr), lambda b: (b, 0)),
        ],
        out_specs=pl.BlockSpec((S, H * Dh), lambda b: (b, 0)),
        out_shape=jax.ShapeDtypeStruct((M, H * Dh), jnp.bfloat16),
        scratch_shapes=[
            pltpu.VMEM((S, D), jnp.bfloat16),
            pltpu.VMEM((S, D), jnp.bfloat16),
        ],
        compiler_params=_CompilerParams(vmem_limit_bytes=_VMEM_LIMIT),
    )(c2, wuk2, wuv2, q2, qr2, kr2)


def kernel(x, Wdkv, Wuk, Wuv, Wq, Wqr, Wkr, Wo):
    x2 = x.reshape(M, D)

    xbf, c2, wuk2, wuv2, q2 = _exchange_q(x2, Wdkv, Wuk, Wuv, Wq)

    qr2, kr2 = _qr_kr(xbf, Wqr, Wkr)

    o2 = _attention(c2, wuk2, wuv2, q2, qr2, kr2)

    out2 = _matmul(o2, Wo, 512, jnp.float32)
    return out2.reshape(B, S, D)


# device time: 177336 ns/iter; 1.0383x vs baseline; 1.0383x over previous
import functools

import jax
import jax.numpy as jnp
from jax import lax
from jax.experimental import pallas as pl
from jax.experimental.pallas import tpu as pltpu

B, S, H, Dh, Dr = 4, 256, 32, 128, 64
D = 4096
DC = 256
DC_SH = 128
M = B * S

_CompilerParams = getattr(pltpu, "CompilerParams", None) or getattr(
    pltpu, "TPUCompilerParams"
)
_VMEM_LIMIT = 100 * 1024 * 1024


_QBLK = 512


def _exq_body(
    x_ref, wdkv_ref, wuk_ref, wuv_ref, wq_ref,
    xbf_ref, c_ref, wuk2_ref, wuv2_ref, q_ref,
    send_sems, recv_sems,
):
    j = pl.program_id(0)
    my_x = lax.axis_index("x")
    my_y = lax.axis_index("y")
    my_z = lax.axis_index("z")
    partner = (1 - my_x, my_y, my_z)

    rdmas = [
        pltpu.make_async_remote_copy(
            src_ref=src,
            dst_ref=dst,
            send_sem=send_sems.at[i],
            recv_sem=recv_sems.at[i],
            device_id=partner,
            device_id_type=pl.DeviceIdType.MESH,
        )
        for i, (src, dst) in enumerate(
            [
                (wuk2_ref.at[pl.ds(0, DC_SH), :], wuk2_ref.at[pl.ds(DC_SH, DC_SH), :]),
                (wuv2_ref.at[pl.ds(0, DC_SH), :], wuv2_ref.at[pl.ds(DC_SH, DC_SH), :]),
                (c_ref.at[:, pl.ds(0, DC_SH)], c_ref.at[:, pl.ds(DC_SH, DC_SH)]),
            ]
        )
    ]

    @pl.when(j == 0)
    def _():
        barrier = pltpu.get_barrier_semaphore()
        pl.semaphore_signal(
            barrier, inc=1, device_id=partner,
            device_id_type=pl.DeviceIdType.MESH,
        )
        pl.semaphore_wait(barrier, 1)

        wuk2_ref[0:DC_SH, :] = wuk_ref[...].astype(jnp.bfloat16)
        wuv2_ref[0:DC_SH, :] = wuv_ref[...].astype(jnp.bfloat16)
        rdmas[0].start()
        rdmas[1].start()
        xbf_ref[...] = x_ref[...].astype(jnp.bfloat16)
        c_ref[:, 0:DC_SH] = jnp.dot(
            xbf_ref[...], wdkv_ref[...].astype(jnp.bfloat16),
            preferred_element_type=jnp.float32,
        ).astype(jnp.bfloat16)
        rdmas[2].start()

    q_ref[...] = jnp.dot(
        xbf_ref[...], wq_ref[...].astype(jnp.bfloat16),
        preferred_element_type=jnp.float32,
    ).astype(jnp.bfloat16)

    @pl.when(j == pl.num_programs(0) - 1)
    def _():
        for rdma in rdmas:
            rdma.wait()


def _exchange_q(x2, wdkv, wuk, wuv, wq):
    return pl.pallas_call(
        _exq_body,
        grid=(D // _QBLK,),
        out_shape=[
            jax.ShapeDtypeStruct((M, D), jnp.bfloat16),
            jax.ShapeDtypeStruct((M, DC), jnp.bfloat16),
            jax.ShapeDtypeStruct((DC, D), jnp.bfloat16),
            jax.ShapeDtypeStruct((DC, D), jnp.bfloat16),
            jax.ShapeDtypeStruct((M, D), jnp.bfloat16),
        ],
        in_specs=[
            pl.BlockSpec((M, D), lambda j: (0, 0)),
            pl.BlockSpec((D, DC_SH), lambda j: (0, 0)),
            pl.BlockSpec((DC_SH, D), lambda j: (0, 0)),
            pl.BlockSpec((DC_SH, D), lambda j: (0, 0)),
            pl.BlockSpec((D, _QBLK), lambda j: (0, j)),
        ],
        out_specs=[
            pl.BlockSpec((M, D), lambda j: (0, 0)),
            pl.BlockSpec((M, DC), lambda j: (0, 0)),
            pl.BlockSpec((DC, D), lambda j: (0, 0)),
            pl.BlockSpec((DC, D), lambda j: (0, 0)),
            pl.BlockSpec((M, _QBLK), lambda j: (0, j)),
        ],
        scratch_shapes=[
            pltpu.SemaphoreType.DMA((3,)),
            pltpu.SemaphoreType.DMA((3,)),
        ],
        compiler_params=_CompilerParams(
            collective_id=0, vmem_limit_bytes=_VMEM_LIMIT
        ),
    )(x2, wdkv, wuk, wuv, wq)


def _matmul_body(a_ref, w_ref, o_ref, *, out_dtype):
    a = a_ref[...]
    w = w_ref[...]
    if a.dtype != jnp.bfloat16:
        a = a.astype(jnp.bfloat16)
    if w.dtype != jnp.bfloat16:
        w = w.astype(jnp.bfloat16)
    o_ref[...] = jnp.dot(a, w, preferred_element_type=jnp.float32).astype(
        out_dtype
    )


def _matmul(a, w, bn, out_dtype):
    m, k = a.shape
    _, n = w.shape
    assert n % bn == 0
    return pl.pallas_call(
        functools.partial(_matmul_body, out_dtype=out_dtype),
        grid=(n // bn,),
        in_specs=[
            pl.BlockSpec((m, k), lambda j: (0, 0)),
            pl.BlockSpec((k, bn), lambda j: (0, j)),
        ],
        out_specs=pl.BlockSpec((m, bn), lambda j: (0, j)),
        out_shape=jax.ShapeDtypeStruct((m, n), out_dtype),
    )(a, w)


def _qr_kr_body(a_ref, wqr_ref, wkr_ref, qr_ref, kr_ref):
    @pl.when(pl.program_id(0) == 0)
    def _():
        kr_ref[...] = jnp.dot(
            a_ref[...], wkr_ref[...].astype(jnp.bfloat16),
            preferred_element_type=jnp.float32,
        ).astype(jnp.bfloat16)

    qr_ref[...] = jnp.dot(
        a_ref[...], wqr_ref[...].astype(jnp.bfloat16),
        preferred_element_type=jnp.float32,
    ).astype(jnp.bfloat16)


def _qr_kr(a, wqr, wkr):
    bn = 512
    n = H * Dr
    return pl.pallas_call(
        _qr_kr_body,
        grid=(n // bn,),
        in_specs=[
            pl.BlockSpec((M, D), lambda j: (0, 0)),
            pl.BlockSpec((D, bn), lambda j: (0, j)),
            pl.BlockSpec((D, Dr), lambda j: (0, 0)),
        ],
        out_specs=[
            pl.BlockSpec((M, bn), lambda j: (0, j)),
            pl.BlockSpec((M, Dr), lambda j: (0, 0)),
        ],
        out_shape=[
            jax.ShapeDtypeStruct((M, n), jnp.bfloat16),
            jax.ShapeDtypeStruct((M, Dr), jnp.bfloat16),
        ],
    )(a, wqr, wkr)


def _attn_body(c_ref, wuk_ref, wuv_ref, q_ref, qr_ref, kr_ref, o_ref,
               k_s, v_s):
    k_s[...] = jnp.dot(
        c_ref[...], wuk_ref[...], preferred_element_type=jnp.float32
    ).astype(jnp.bfloat16)
    v_s[...] = jnp.dot(
        c_ref[...], wuv_ref[...], preferred_element_type=jnp.float32
    ).astype(jnp.bfloat16)

    kr = kr_ref[...]
    kr_cat = jnp.concatenate([kr, kr], axis=1)
    lane = lax.broadcasted_iota(jnp.int32, (S, 2 * Dr), 1)
    scale = (Dh + Dr) ** -0.5
    masks = [(lane // Dr) == 0, (lane // Dr) == 1]
    dims = (((1,), (1,)), ((), ()))
    for h in range(H):
        q = (q_ref[:, h * Dh:(h + 1) * Dh] * scale).astype(jnp.bfloat16)
        k = k_s[:, h * Dh:(h + 1) * Dh]
        v = v_s[:, h * Dh:(h + 1) * Dh]
        qrp = qr_ref[:, (h // 2) * 2 * Dr:(h // 2 + 1) * 2 * Dr]
        qr_m = jnp.where(masks[h % 2], qrp * scale, 0).astype(jnp.bfloat16)
        s = lax.dot_general(q, k, dims, preferred_element_type=jnp.float32)
        s = s + lax.dot_general(
            qr_m, kr_cat, dims, preferred_element_type=jnp.float32
        )
        p = jnp.exp(s)
        denom = jnp.sum(p, axis=1, keepdims=True)
        o = jnp.dot(
            p.astype(jnp.bfloat16), v, preferred_element_type=jnp.float32
        )
        o_ref[:, h * Dh:(h + 1) * Dh] = (o * (1.0 / denom)).astype(
            jnp.bfloat16
        )


def _attention(c2, wuk2, wuv2, q2, qr2, kr2):
    return pl.pallas_call(
        _attn_body,
        grid=(B,),
        in_specs=[
            pl.BlockSpec((S, DC), lambda b: (b, 0)),
            pl.BlockSpec((DC, D), lambda b: (0, 0)),
            pl.BlockSpec((DC, D), lambda b: (0, 0)),
            pl.BlockSpec((S, D), lambda b: (b, 0)),
            pl.BlockSpec((S, H * Dr), lambda b: (b, 0)),
            pl.BlockSpec((S, Dr), lambda b: (b, 0)),
        ],
        out_specs=pl.BlockSpec((S, H * Dh), lambda b: (b, 0)),
        out_shape=jax.ShapeDtypeStruct((M, H * Dh), jnp.bfloat16),
        scratch_shapes=[
            pltpu.VMEM((S, D), jnp.bfloat16),
            pltpu.VMEM((S, D), jnp.bfloat16),
        ],
    )(c2, wuk2, wuv2, q2, qr2, kr2)


def kernel(x, Wdkv, Wuk, Wuv, Wq, Wqr, Wkr, Wo):
    x2 = x.reshape(M, D)

    xbf, c2, wuk2, wuv2, q2 = _exchange_q(x2, Wdkv, Wuk, Wuv, Wq)

    qr2, kr2 = _qr_kr(xbf, Wqr, Wkr)

    o2 = _attention(c2, wuk2, wuv2, q2, qr2, kr2)

    out2 = _matmul(o2, Wo, 512, jnp.float32)
    return out2.reshape(B, S, D)
